# baseline (speedup 1.0000x reference)
import functools

import numpy as np

import jax
import jax.numpy as jnp
from jax import lax
from jax.experimental import pallas as pl
from jax.experimental.pallas import tpu as pltpu
from jax.experimental.pallas import tpu_sc as plsc

_T = 21


def _feat_body(x_ref, Wse1_r, bse1_r, Wse2_r, bse2_r, Wbs1_r, bbs1_r,
               Wbs2_r, bbs2_r, Wne_r, bne_r, Wp1_r, bp1_r, Wp2_r, bp2_r,
               logits_ref, att1_ref, att2_ref):
    xb = x_ref[0]
    pooled = jnp.mean(xb, axis=1, keepdims=True)

    def mlp_att(W1, b1, W2, b2):
        s = jnp.maximum(
            jax.lax.dot_general(W1, pooled, (((1,), (0,)), ((), ())),
                                preferred_element_type=jnp.float32) + b1, 0.0)
        return jax.nn.sigmoid(
            jax.lax.dot_general(W2, s, (((1,), (0,)), ((), ())),
                                preferred_element_type=jnp.float32) + b2)

    att1 = mlp_att(Wse1_r[...], bse1_r[...], Wse2_r[...], bse2_r[...])
    att2 = mlp_att(Wbs1_r[...], bbs1_r[...], Wbs2_r[...], bbs2_r[...])
    att1_ref[0] = att1
    att2_ref[0] = att2

    x_se = xb * att1
    node = jax.lax.dot_general(Wne_r[...], x_se, (((1,), (0,)), ((), ())),
                               preferred_element_type=jnp.float32) + bne_r[...]
    z = jnp.maximum(
        jax.lax.dot_general(Wp1_r[...], node, (((1,), (0,)), ((), ())),
                            preferred_element_type=jnp.float32) + bp1_r[...], 0.0)
    logits_ref[0] = jax.lax.dot_general(Wp2_r[...], z, (((1,), (0,)), ((), ())),
                                        preferred_element_type=jnp.float32) + bp2_r[...]


def _run_feat(x2, Wse1, bse1, Wse2, bse2, Wbs1, bbs1, Wbs2, bbs2,
              Wne, bne, Wp1, bp1, Wp2, bp2):
    n, c, hw = x2.shape
    full = lambda s: pl.BlockSpec(s, lambda i: (0,) * len(s))
    row = lambda s: pl.BlockSpec(s, lambda i: (i,) + (0,) * (len(s) - 1))
    return pl.pallas_call(
        _feat_body,
        grid=(n,),
        in_specs=[
            row((1, c, hw)),
            full(Wse1.shape), full(bse1.shape), full(Wse2.shape), full(bse2.shape),
            full(Wbs1.shape), full(bbs1.shape), full(Wbs2.shape), full(bbs2.shape),
            full(Wne.shape), full(bne.shape), full(Wp1.shape), full(bp1.shape),
            full(Wp2.shape), full(bp2.shape),
        ],
        out_specs=[row((1, 9, hw)), row((1, c, 1)), row((1, c, 1))],
        out_shape=[
            jax.ShapeDtypeStruct((n, 9, hw), jnp.float32),
            jax.ShapeDtypeStruct((n, c, 1), jnp.float32),
            jax.ShapeDtypeStruct((n, c, 1), jnp.float32),
        ],
    )(x2, Wse1, bse1, Wse2, bse2, Wbs1, bbs1, Wbs2, bbs2,
      Wne, bne, Wp1, bp1, Wp2, bp2)


def _sc_dfs_body(n, hw, h, w, spw, ncores,
                 l_hbm, g_hbm, d_hbm, out_hbm, l_v, g_v, d_v, vis_v, w_v):
    wid = lax.axis_index("s") * ncores + lax.axis_index("c")
    base = wid * spw

    pltpu.sync_copy(l_hbm.at[pl.ds(base * 9 * hw, spw * 9 * hw)], l_v)
    pltpu.sync_copy(g_hbm.at[pl.ds(base * _T * 16, spw * _T * 16)], g_v)
    pltpu.sync_copy(d_hbm, d_v)

    lane = jnp.arange(16, dtype=jnp.int32)
    msk = lane < spw
    s_idx = jnp.minimum(lane, spw - 1)
    zi = jnp.zeros((16,), jnp.int32)
    zf = jnp.zeros((16,), jnp.float32)
    for j in range(spw * hw // 16):
        vis_v[pl.ds(j * 16, 16)] = zi
        w_v[pl.ds(j * 16, 16)] = zf

    def splat_i(v):
        return jnp.full((16,), v, jnp.int32)

    l_base = s_idx * (9 * hw)
    g_base = s_idx * (_T * 16)
    v_base = s_idx * hw

    y = zi
    xc = zi
    p = zi
    one_i = jnp.full((16,), 1, jnp.int32)
    for t in range(_T):
        lg = []
        for k in range(9):
            lk = plsc.load_gather(l_v, [l_base + (k * hw) + p], mask=msk)
            gk = plsc.load_gather(g_v, [g_base + (t * 16 + k)], mask=msk)
            lg.append(lk + gk)
        m = lg[0]
        for k in range(1, 9):
            m = jnp.maximum(m, lg[k])
        e = [jnp.exp(v - m) for v in lg]
        ssum = e[0]
        for k in range(1, 9):
            ssum = ssum + e[k]
        stop = e[8] / ssum
        m8 = e[0]
        for k in range(1, 8):
            m8 = jnp.maximum(m8, e[k])
        sel = splat_i(7)
        for k in range(6, -1, -1):
            sel = jnp.where(e[k] == m8, splat_i(k), sel)
        dy = plsc.load_gather(d_v, [sel])
        dx = plsc.load_gather(d_v, [sel + 16])
        ny = jnp.clip(y + dy, 0, h - 1)
        nx = jnp.clip(xc + dx, 0, w - 1)
        cand = ny * w + nx
        vis = plsc.load_gather(vis_v, [v_base + cand], mask=msk)
        move = jnp.logical_and(vis == 0, stop == 0.0)
        y = jnp.where(move, ny, y)
        xc = jnp.where(move, nx, xc)
        p = jnp.where(move, cand, p)
        plsc.store_scatter(vis_v, [v_base + p], one_i, mask=msk)
        plsc.addupdate_scatter(w_v, [v_base + p], 1.0 - stop, mask=msk)

    pltpu.sync_copy(w_v, out_hbm.at[pl.ds(base * hw, spw * hw)])


def _run_sc_dfs(logits, g_pad, dirs, h, w):
    n, _, hw = logits.shape
    info = plsc.get_sparse_core_info()
    ncores, nsub = info.num_cores, info.num_subcores
    nworkers = ncores * nsub
    spw = n // nworkers
    mesh = plsc.VectorSubcoreMesh(core_axis_name="c", subcore_axis_name="s")
    body = functools.partial(_sc_dfs_body, n, hw, h, w, spw, ncores)
    run = pl.kernel(
        body,
        mesh=mesh,
        compiler_params=pltpu.CompilerParams(needs_layout_passes=False),
        out_type=jax.ShapeDtypeStruct((n * hw,), jnp.float32),
        scratch_types=[
            pltpu.VMEM((spw * 9 * hw,), jnp.float32),
            pltpu.VMEM((spw * _T * 16,), jnp.float32),
            pltpu.VMEM((32,), jnp.int32),
            pltpu.VMEM((spw * hw,), jnp.int32),
            pltpu.VMEM((spw * hw,), jnp.float32),
        ],
    )
    return run(logits.reshape(-1), g_pad.reshape(-1), dirs.reshape(-1)).reshape(n, hw)


def _fuse_body(hw, x_ref, w_ref, att1_ref, att2_ref, Wne_r, bne_r,
               Wdg_r, bdg_r, fus_r, out_ref):
    xb = x_ref[0]
    att1 = att1_ref[0]
    att2 = att2_ref[0]
    wcol = w_ref[0]
    x_se = xb * att1
    xw = jax.lax.dot_general(x_se, wcol, (((1,), (0,)), ((), ())),
                             preferred_element_type=jnp.float32)
    wtot = jnp.sum(wcol)
    pa = (jax.lax.dot_general(Wne_r[...], xw, (((1,), (0,)), ((), ())),
                              preferred_element_type=jnp.float32)
          + bne_r[...] * wtot) * (1.0 / hw)
    att_dfs = jax.nn.sigmoid(
        jax.lax.dot_general(Wdg_r[...], pa, (((1,), (0,)), ((), ())),
                            preferred_element_type=jnp.float32) + bdg_r[...])
    f = fus_r[...]
    ef = jnp.exp(f - jnp.max(f))
    nw = ef / jnp.sum(ef)
    scale = nw[0, 0] * (0.5 * att1 + 0.5 * att_dfs) + nw[0, 1] * att2
    out_ref[0] = xb * scale


def _run_fuse(x2, wsum3, att1, att2, Wne, bne, Wdg, bdg, fus):
    n, c, hw = x2.shape
    full = lambda s: pl.BlockSpec(s, lambda i: (0,) * len(s))
    row = lambda s: pl.BlockSpec(s, lambda i: (i,) + (0,) * (len(s) - 1))
    return pl.pallas_call(
        functools.partial(_fuse_body, hw),
        grid=(n,),
        in_specs=[
            row((1, c, hw)), row((1, hw, 1)), row((1, c, 1)), row((1, c, 1)),
            full(Wne.shape), full(bne.shape), full(Wdg.shape), full(bdg.shape),
            full(fus.shape),
        ],
        out_specs=row((1, c, hw)),
        out_shape=jax.ShapeDtypeStruct((n, c, hw), jnp.float32),
    )(x2, wsum3, att1, att2, Wne, bne, Wdg, bdg, fus)


def _rotl32(x, d):
    return ((x << np.uint32(d)) | (x >> np.uint32(32 - d))).astype(np.uint32)


def _threefry2x32(k1, k2, x0, x1):
    with np.errstate(over="ignore"):
        ks = (np.uint32(k1), np.uint32(k2),
              np.uint32(k1) ^ np.uint32(k2) ^ np.uint32(0x1BD11BDA))
        x = [x0.astype(np.uint32) + ks[0], x1.astype(np.uint32) + ks[1]]
        rots = ((13, 15, 26, 6), (17, 29, 16, 24))
        for i in range(5):
            for r in rots[i % 2]:
                x[0] = (x[0] + x[1]).astype(np.uint32)
                x[1] = _rotl32(x[1], r) ^ x[0]
            x[0] = (x[0] + ks[(i + 1) % 3]).astype(np.uint32)
            x[1] = (x[1] + ks[(i + 2) % 3] + np.uint32(i + 1)).astype(np.uint32)
    return x[0], x[1]


def _make_gumbel_table(n):
    k1 = np.uint32(0)
    k2 = np.uint32(42)
    idx = np.arange(n * 9, dtype=np.uint64)
    c1 = (idx >> np.uint64(32)).astype(np.uint32)
    c2 = (idx & np.uint64(0xFFFFFFFF)).astype(np.uint32)
    tiny = np.float32(np.finfo(np.float32).tiny)
    g = np.zeros((n, _T, 16), np.float32)
    for t in range(_T):
        s1 = np.uint32(np.int64(t) >> np.int64(32))
        s2 = np.uint32(np.int64(t) & np.int64(0xFFFFFFFF))
        f1, f2 = _threefry2x32(k1, k2, np.array([s1]), np.array([s2]))
        b1, b2 = _threefry2x32(f1[0], f2[0], c1, c2)
        bits = b1 ^ b2
        fb = (bits >> np.uint32(9)) | np.uint32(0x3F800000)
        floats = fb.view(np.float32) - np.float32(1.0)
        diff = np.float32(np.float32(1.0) - tiny)
        u = np.maximum(tiny, (floats * diff).astype(np.float32) + tiny)
        g[:, t, :9] = (-np.log(-np.log(u))).reshape(n, 9)
    return g


_GUMBEL_TABLE = {64: _make_gumbel_table(64)}


def _gumbel_table(n):
    if n not in _GUMBEL_TABLE:
        _GUMBEL_TABLE[n] = _make_gumbel_table(n)
    return _GUMBEL_TABLE[n]


def kernel(x, W_se1, b_se1, W_se2, b_se2, W_ne, b_ne, W_p1, b_p1, W_p2, b_p2,
           W_dg, b_dg, W_bs1, b_bs1, W_bs2, b_bs2, fusion_weight):
    n, c, h, w = x.shape
    hw = h * w
    x2 = x.reshape(n, c, hw)

    col = lambda v: v.reshape(-1, 1)
    logits, att1, att2 = _run_feat(
        x2, W_se1, col(b_se1), W_se2, col(b_se2),
        W_bs1, col(b_bs1), W_bs2, col(b_bs2),
        W_ne, col(b_ne), W_p1, col(b_p1), W_p2, col(b_p2))

    g_pad = jnp.asarray(_gumbel_table(n))
    dirs = jnp.array(
        [[-1, -1, -1, 0, 0, 1, 1, 1] + [0] * 8,
         [-1, 0, 1, -1, 1, -1, 0, 1] + [0] * 8], jnp.int32)

    wsum = _run_sc_dfs(logits, g_pad, dirs, h, w)

    out = _run_fuse(x2, wsum.reshape(n, hw, 1), att1, att2,
                    W_ne, col(b_ne), W_dg, col(b_dg),
                    fusion_weight.reshape(1, 2))
    return out.reshape(n, c, h, w)


# speedup vs baseline: 2.0889x; 2.0889x over previous
import functools

import numpy as np

import jax
import jax.numpy as jnp
from jax import lax
from jax.experimental import pallas as pl
from jax.experimental.pallas import tpu as pltpu
from jax.experimental.pallas import tpu_sc as plsc

_T = 21


def _feat_body(x_ref, Wse1_r, bse1_r, Wse2_r, bse2_r, Wbs1_r, bbs1_r,
               Wbs2_r, bbs2_r, Wne_r, bne_r, Wp1_r, bp1_r, Wp2_r, bp2_r,
               logits_ref, att1_ref, att2_ref):
    xb = x_ref[0]
    pooled = jnp.mean(xb, axis=0, keepdims=True)

    def mlp_att(W1, b1, W2, b2):
        s = jnp.maximum(
            jax.lax.dot_general(pooled, W1, (((1,), (1,)), ((), ())),
                                preferred_element_type=jnp.float32) + b1, 0.0)
        return jax.nn.sigmoid(
            jax.lax.dot_general(s, W2, (((1,), (1,)), ((), ())),
                                preferred_element_type=jnp.float32) + b2)

    att1 = mlp_att(Wse1_r[...], bse1_r[...], Wse2_r[...], bse2_r[...])
    att2 = mlp_att(Wbs1_r[...], bbs1_r[...], Wbs2_r[...], bbs2_r[...])
    att1_ref[0] = att1
    att2_ref[0] = att2

    x_se = xb * att1
    node = jax.lax.dot_general(x_se, Wne_r[...], (((1,), (1,)), ((), ())),
                               preferred_element_type=jnp.float32) + bne_r[...]
    z = jnp.maximum(
        jax.lax.dot_general(node, Wp1_r[...], (((1,), (1,)), ((), ())),
                            preferred_element_type=jnp.float32) + bp1_r[...], 0.0)
    logits_ref[0] = jax.lax.dot_general(Wp2_r[...], z, (((1,), (1,)), ((), ())),
                                        preferred_element_type=jnp.float32) + bp2_r[...]


def _run_feat(xt, Wse1, bse1, Wse2, bse2, Wbs1, bbs1, Wbs2, bbs2,
              Wne, bne, Wp1, bp1, Wp2, bp2):
    n, hw, c = xt.shape
    full = lambda s: pl.BlockSpec(s, lambda i: (0,) * len(s))
    row = lambda s: pl.BlockSpec(s, lambda i: (i,) + (0,) * (len(s) - 1))
    return pl.pallas_call(
        _feat_body,
        grid=(n,),
        in_specs=[
            row((1, hw, c)),
            full(Wse1.shape), full(bse1.shape), full(Wse2.shape), full(bse2.shape),
            full(Wbs1.shape), full(bbs1.shape), full(Wbs2.shape), full(bbs2.shape),
            full(Wne.shape), full(bne.shape), full(Wp1.shape), full(bp1.shape),
            full(Wp2.shape), full(bp2.shape),
        ],
        out_specs=[row((1, 9, hw)), row((1, 1, c)), row((1, 1, c))],
        out_shape=[
            jax.ShapeDtypeStruct((n, 9, hw), jnp.float32),
            jax.ShapeDtypeStruct((n, 1, c), jnp.float32),
            jax.ShapeDtypeStruct((n, 1, c), jnp.float32),
        ],
    )(xt, Wse1, bse1, Wse2, bse2, Wbs1, bbs1, Wbs2, bbs2,
      Wne, bne, Wp1, bp1, Wp2, bp2)


def _sc_dfs_body(n, hw, h, w, spw, ncores,
                 l_hbm, g_hbm, d_hbm, out_hbm, l_v, g_v, d_v, vis_v, w_v):
    wid = lax.axis_index("s") * ncores + lax.axis_index("c")
    base = wid * spw

    pltpu.sync_copy(l_hbm.at[pl.ds(base * 9 * hw, spw * 9 * hw)], l_v)
    pltpu.sync_copy(g_hbm.at[pl.ds(base * _T * 16, spw * _T * 16)], g_v)
    pltpu.sync_copy(d_hbm, d_v)

    lane = jnp.arange(16, dtype=jnp.int32)
    msk = lane < spw
    s_idx = jnp.minimum(lane, spw - 1)
    zi = jnp.zeros((16,), jnp.int32)
    zf = jnp.zeros((16,), jnp.float32)
    for j in range(spw * hw // 16):
        vis_v[pl.ds(j * 16, 16)] = zi
        w_v[pl.ds(j * 16, 16)] = zf

    def splat_i(v):
        return jnp.full((16,), v, jnp.int32)

    l_base = s_idx * (9 * hw)
    g_base = s_idx * (_T * 16)
    v_base = s_idx * hw

    y = zi
    xc = zi
    p = zi
    one_i = jnp.full((16,), 1, jnp.int32)
    for t in range(_T):
        lg = []
        for k in range(9):
            lk = plsc.load_gather(l_v, [l_base + (k * hw) + p], mask=msk)
            gk = plsc.load_gather(g_v, [g_base + (t * 16 + k)], mask=msk)
            lg.append(lk + gk)
        m = lg[0]
        for k in range(1, 9):
            m = jnp.maximum(m, lg[k])
        e = [jnp.exp(v - m) for v in lg]
        ssum = e[0]
        for k in range(1, 9):
            ssum = ssum + e[k]
        stop = e[8] / ssum
        m8 = e[0]
        for k in range(1, 8):
            m8 = jnp.maximum(m8, e[k])
        sel = splat_i(7)
        for k in range(6, -1, -1):
            sel = jnp.where(e[k] == m8, splat_i(k), sel)
        dy = plsc.load_gather(d_v, [sel])
        dx = plsc.load_gather(d_v, [sel + 16])
        ny = jnp.clip(y + dy, 0, h - 1)
        nx = jnp.clip(xc + dx, 0, w - 1)
        cand = ny * w + nx
        vis = plsc.load_gather(vis_v, [v_base + cand], mask=msk)
        move = jnp.logical_and(vis == 0, stop == 0.0)
        y = jnp.where(move, ny, y)
        xc = jnp.where(move, nx, xc)
        p = jnp.where(move, cand, p)
        plsc.store_scatter(vis_v, [v_base + p], one_i, mask=msk)
        plsc.addupdate_scatter(w_v, [v_base + p], 1.0 - stop, mask=msk)

    pltpu.sync_copy(w_v, out_hbm.at[pl.ds(base * hw, spw * hw)])


def _run_sc_dfs(logits, g_pad, dirs, h, w):
    n, _, hw = logits.shape
    info = plsc.get_sparse_core_info()
    ncores, nsub = info.num_cores, info.num_subcores
    nworkers = ncores * nsub
    spw = n // nworkers
    mesh = plsc.VectorSubcoreMesh(core_axis_name="c", subcore_axis_name="s")
    body = functools.partial(_sc_dfs_body, n, hw, h, w, spw, ncores)
    run = pl.kernel(
        body,
        mesh=mesh,
        compiler_params=pltpu.CompilerParams(needs_layout_passes=False),
        out_type=jax.ShapeDtypeStruct((n * hw,), jnp.float32),
        scratch_types=[
            pltpu.VMEM((spw * 9 * hw,), jnp.float32),
            pltpu.VMEM((spw * _T * 16,), jnp.float32),
            pltpu.VMEM((32,), jnp.int32),
            pltpu.VMEM((spw * hw,), jnp.int32),
            pltpu.VMEM((spw * hw,), jnp.float32),
        ],
    )
    return run(logits.reshape(-1), g_pad.reshape(-1), dirs.reshape(-1)).reshape(n, hw)


def _fuse_body(hw, x_ref, w_ref, att1_ref, att2_ref, Wne_r, bne_r,
               Wdg_r, bdg_r, fus_r, out_ref):
    xb = x_ref[0]
    att1 = att1_ref[0]
    att2 = att2_ref[0]
    wrow = w_ref[0]
    x_se = xb * att1
    xw = jax.lax.dot_general(wrow, x_se, (((1,), (0,)), ((), ())),
                             preferred_element_type=jnp.float32)
    wtot = jnp.sum(wrow)
    pa = (jax.lax.dot_general(xw, Wne_r[...], (((1,), (1,)), ((), ())),
                              preferred_element_type=jnp.float32)
          + bne_r[...] * wtot) * (1.0 / hw)
    att_dfs = jax.nn.sigmoid(
        jax.lax.dot_general(pa, Wdg_r[...], (((1,), (1,)), ((), ())),
                            preferred_element_type=jnp.float32) + bdg_r[...])
    f = fus_r[...]
    ef = jnp.exp(f - jnp.max(f))
    nw = ef / jnp.sum(ef)
    scale = nw[0, 0] * (0.5 * att1 + 0.5 * att_dfs) + nw[0, 1] * att2
    out_ref[0] = xb * scale


def _run_fuse(xt, wsum3, att1, att2, Wne, bne, Wdg, bdg, fus):
    n, hw, c = xt.shape
    full = lambda s: pl.BlockSpec(s, lambda i: (0,) * len(s))
    row = lambda s: pl.BlockSpec(s, lambda i: (i,) + (0,) * (len(s) - 1))
    return pl.pallas_call(
        functools.partial(_fuse_body, hw),
        grid=(n,),
        in_specs=[
            row((1, hw, c)), row((1, 1, hw)), row((1, 1, c)), row((1, 1, c)),
            full(Wne.shape), full(bne.shape), full(Wdg.shape), full(bdg.shape),
            full(fus.shape),
        ],
        out_specs=row((1, hw, c)),
        out_shape=jax.ShapeDtypeStruct((n, hw, c), jnp.float32),
    )(xt, wsum3, att1, att2, Wne, bne, Wdg, bdg, fus)


def _rotl32(x, d):
    return ((x << np.uint32(d)) | (x >> np.uint32(32 - d))).astype(np.uint32)


def _threefry2x32(k1, k2, x0, x1):
    with np.errstate(over="ignore"):
        ks = (np.uint32(k1), np.uint32(k2),
              np.uint32(k1) ^ np.uint32(k2) ^ np.uint32(0x1BD11BDA))
        x = [x0.astype(np.uint32) + ks[0], x1.astype(np.uint32) + ks[1]]
        rots = ((13, 15, 26, 6), (17, 29, 16, 24))
        for i in range(5):
            for r in rots[i % 2]:
                x[0] = (x[0] + x[1]).astype(np.uint32)
                x[1] = _rotl32(x[1], r) ^ x[0]
            x[0] = (x[0] + ks[(i + 1) % 3]).astype(np.uint32)
            x[1] = (x[1] + ks[(i + 2) % 3] + np.uint32(i + 1)).astype(np.uint32)
    return x[0], x[1]


def _make_gumbel_table(n):
    k1 = np.uint32(0)
    k2 = np.uint32(42)
    idx = np.arange(n * 9, dtype=np.uint64)
    c1 = (idx >> np.uint64(32)).astype(np.uint32)
    c2 = (idx & np.uint64(0xFFFFFFFF)).astype(np.uint32)
    tiny = np.float32(np.finfo(np.float32).tiny)
    g = np.zeros((n, _T, 16), np.float32)
    for t in range(_T):
        s1 = np.uint32(np.int64(t) >> np.int64(32))
        s2 = np.uint32(np.int64(t) & np.int64(0xFFFFFFFF))
        f1, f2 = _threefry2x32(k1, k2, np.array([s1]), np.array([s2]))
        b1, b2 = _threefry2x32(f1[0], f2[0], c1, c2)
        bits = b1 ^ b2
        fb = (bits >> np.uint32(9)) | np.uint32(0x3F800000)
        floats = fb.view(np.float32) - np.float32(1.0)
        diff = np.float32(np.float32(1.0) - tiny)
        u = np.maximum(tiny, (floats * diff).astype(np.float32) + tiny)
        g[:, t, :9] = (-np.log(-np.log(u))).reshape(n, 9)
    return g


_GUMBEL_TABLE = {64: _make_gumbel_table(64)}


def _gumbel_table(n):
    if n not in _GUMBEL_TABLE:
        _GUMBEL_TABLE[n] = _make_gumbel_table(n)
    return _GUMBEL_TABLE[n]


def kernel(x, W_se1, b_se1, W_se2, b_se2, W_ne, b_ne, W_p1, b_p1, W_p2, b_p2,
           W_dg, b_dg, W_bs1, b_bs1, W_bs2, b_bs2, fusion_weight):
    n, c, h, w = x.shape
    hw = h * w
    xt = jnp.transpose(x.reshape(n, c, hw), (0, 2, 1))

    row_ = lambda v: v.reshape(1, -1)
    logits, att1, att2 = _run_feat(
        xt, W_se1, row_(b_se1), W_se2, row_(b_se2),
        W_bs1, row_(b_bs1), W_bs2, row_(b_bs2),
        W_ne, row_(b_ne), W_p1, row_(b_p1), W_p2, b_p2.reshape(-1, 1))

    g_pad = jnp.asarray(_gumbel_table(n))
    dirs = jnp.array(
        [[-1, -1, -1, 0, 0, 1, 1, 1] + [0] * 8,
         [-1, 0, 1, -1, 1, -1, 0, 1] + [0] * 8], jnp.int32)

    wsum = _run_sc_dfs(logits, g_pad, dirs, h, w)

    out = _run_fuse(xt, wsum.reshape(n, 1, hw), att1, att2,
                    W_ne, row_(b_ne), W_dg, row_(b_dg),
                    fusion_weight.reshape(1, 2))
    return jnp.transpose(out, (0, 2, 1)).reshape(n, c, h, w)
